# baseline (speedup 1.0000x reference)
import jax
import jax.numpy as jnp
from jax import lax
from jax.experimental import pallas as pl
from jax.experimental.pallas import tpu as pltpu
from jax.experimental.pallas import tpu_sc as plsc

N = 10000
E = 320000
D_IN = 128
D_HID = 256
N_CLASSES = 41
CP = 48

NC = 2
NS = 16
NW = NC * NS

ROWS_PT = N // NS
RB = 1000

CHUNK = 128
EROWS = E // CHUNK
RPT = EROWS // NW
XTRA = EROWS - RPT * NW
NPAIR = RPT // 2


def _sc_agg(width, with_count):
  mesh = plsc.VectorSubcoreMesh(
      core_axis_name="c", subcore_axis_name="s", num_cores=NC, num_subcores=NS)

  out_type = [jax.ShapeDtypeStruct((NC, N, width), jnp.float32)]
  scratch = [
      pltpu.VMEM((4, CHUNK), jnp.int32),
      pltpu.VMEM((4, CHUNK), jnp.int32),
      pltpu.VMEM((CHUNK, width), jnp.float32),
      pltpu.VMEM((CHUNK, width), jnp.float32),
      pltpu.VMEM_SHARED((N, width), jnp.float32),
      pltpu.SemaphoreType.DMA,
      pltpu.SemaphoreType.DMA,
      pltpu.SemaphoreType.DMA,
      pltpu.SemaphoreType.DMA,
      pltpu.SemaphoreType.DMA,
  ]
  if with_count:
    out_type.append(jax.ShapeDtypeStruct((NC, N, 16), jnp.float32))
    scratch += [
        pltpu.VMEM((CHUNK, 16), jnp.float32),
        pltpu.VMEM((CHUNK, 16), jnp.float32),
        pltpu.VMEM_SHARED((N, 16), jnp.float32),
    ]
  WL = width // 16

  def body(feat_hbm, edges_hbm, *rest):
    if with_count:
      (sum_hbm, cnt_hbm, sidx, didx, rows_a, rows_b, acc_sh, sem_a, sem_b,
       sem_i, sem_sa, sem_sb, ones_v, z16, cnt_sh) = rest
    else:
      (sum_hbm, sidx, didx, rows_a, rows_b, acc_sh, sem_a, sem_b,
       sem_i, sem_sa, sem_sb) = rest

    c = lax.axis_index("c")
    s = lax.axis_index("s")
    tile = c * NS + s
    base = tile * RPT
    has_extra = tile < XTRA

    def erow(j):
      return jnp.where(j == RPT, EROWS - XTRA + tile,
                       jnp.minimum(base + j, EROWS - 1))

    def idx_load(j, start):
      for ring, half in ((sidx, 0), (didx, 1)):
        cp = pltpu.make_async_copy(edges_hbm.at[half, erow(j)],
                                   ring.at[j % 4], sem_i)
        if start:
          cp.start()
        else:
          cp.wait()

    def scat(rows, sem, slot, start):
      cp = pltpu.make_async_copy(rows, acc_sh.at[didx.at[slot]], sem)
      cp.start(add=True) if start else cp.wait()
      if with_count:
        cp2 = pltpu.make_async_copy(ones_v, cnt_sh.at[didx.at[slot]], sem)
        cp2.start(add=True) if start else cp2.wait()

    with jax.named_scope("agg_prologue"):
      def zrow(i, _):
        r = i // WL
        q = (i % WL) * 16
        rows_a[r, pl.ds(q, 16)] = jnp.zeros((16,), jnp.float32)
        return _
      lax.fori_loop(0, CHUNK * WL, zrow, None)
      if with_count:
        def f16(r, _):
          ones_v[r] = jnp.ones((16,), jnp.float32)
          z16[r] = jnp.zeros((16,), jnp.float32)
          return _
        lax.fori_loop(0, CHUNK, f16, None)

      row0 = s * ROWS_PT
      full, part = divmod(ROWS_PT, CHUNK)
      for k in range(full):
        pltpu.sync_copy(rows_a, acc_sh.at[pl.ds(row0 + k * CHUNK, CHUNK)])
        if with_count:
          pltpu.sync_copy(z16, cnt_sh.at[pl.ds(row0 + k * CHUNK, CHUNK)])
      if part:
        pltpu.sync_copy(rows_a.at[pl.ds(0, part)],
                        acc_sh.at[pl.ds(row0 + full * CHUNK, part)])
        if with_count:
          pltpu.sync_copy(z16.at[pl.ds(0, part)],
                          cnt_sh.at[pl.ds(row0 + full * CHUNK, part)])

      for j in range(2):
        pltpu.sync_copy(edges_hbm.at[0, erow(j)], sidx.at[j])
        pltpu.sync_copy(edges_hbm.at[1, erow(j)], didx.at[j])
      idx_load(2, start=True)
      idx_load(3, start=True)
      plsc.subcore_barrier()

    with jax.named_scope("agg_edges"):
      pltpu.async_copy(feat_hbm.at[sidx.at[0]], rows_a, sem_a)
      pltpu.async_copy(feat_hbm.at[sidx.at[1]], rows_b, sem_b)

      def pair(i, _):
        j0 = 2 * i
        s0 = j0 % 4
        s1 = (j0 + 1) % 4

        pltpu.make_async_copy(feat_hbm.at[sidx.at[s0]], rows_a, sem_a).wait()
        scat(rows_a, sem_sa, s0, start=True)
        pltpu.make_async_copy(feat_hbm.at[sidx.at[s1]], rows_b, sem_b).wait()
        scat(rows_b, sem_sb, s1, start=True)

        scat(rows_a, sem_sa, s0, start=False)
        idx_load(j0 + 2, start=False)
        idx_load(j0 + 3, start=False)
        idx_load(j0 + 4, start=True)

        @pl.when((i < NPAIR - 1) | has_extra)
        def _():
          pltpu.async_copy(feat_hbm.at[sidx.at[(j0 + 2) % 4]], rows_a, sem_a)

        scat(rows_b, sem_sb, s1, start=False)
        idx_load(j0 + 5, start=True)

        @pl.when(i < NPAIR - 1)
        def _():
          pltpu.async_copy(feat_hbm.at[sidx.at[(j0 + 3) % 4]], rows_b, sem_b)
        return _
      lax.fori_loop(0, NPAIR, pair, None)

      idx_load(2 * NPAIR + 2, start=False)
      idx_load(2 * NPAIR + 3, start=False)

      @pl.when(has_extra)
      def _():
        pltpu.make_async_copy(feat_hbm.at[sidx.at[RPT % 4]], rows_a,
                              sem_a).wait()
        scat(rows_a, sem_sa, RPT % 4, start=True)
        scat(rows_a, sem_sa, RPT % 4, start=False)
      plsc.subcore_barrier()

    with jax.named_scope("agg_writeback"):
      pltpu.sync_copy(acc_sh.at[pl.ds(row0, ROWS_PT)],
                      sum_hbm.at[c, pl.ds(row0, ROWS_PT)])
      if with_count:
        pltpu.sync_copy(cnt_sh.at[pl.ds(row0, ROWS_PT)],
                        cnt_hbm.at[c, pl.ds(row0, ROWS_PT)])

  return pl.kernel(
      body, out_type=out_type, mesh=mesh, scratch_types=scratch,
      compiler_params=pltpu.CompilerParams(use_tc_tiling_on_sc=False))


_sc_agg_feat = _sc_agg(D_IN, with_count=True)
_sc_agg_proj = _sc_agg(CP, with_count=False)


def _tc_fused(x, sums, cnts, ws1, wn1, b1, ws2, wn2, b2):
  def body(x_r, s0_r, s1_r, c0_r, c1_r, ws1_r, wn1_r, b1_r, ws2_r, wn2_r,
           b2_r, p_r, s_r):
    deg = c0_r[0][:, 0:1] + c1_r[0][:, 0:1]
    dinv = 1.0 / jnp.maximum(deg, 1.0)
    a = (s0_r[0] + s1_r[0]) * dinv
    h = (jnp.dot(x_r[...], ws1_r[...], preferred_element_type=jnp.float32)
         + jnp.dot(a, wn1_r[...], preferred_element_type=jnp.float32)
         + b1_r[...])
    h = jnp.maximum(h, 0.0)
    p_r[...] = jnp.dot(h, wn2_r[...], preferred_element_type=jnp.float32)
    s_r[...] = (jnp.dot(h, ws2_r[...], preferred_element_type=jnp.float32)
                + b2_r[...])

  row = lambda i: (i, 0)
  fix = lambda i: (0, 0)
  core0 = lambda i: (0, i, 0)
  core1 = lambda i: (1, i, 0)
  return pl.pallas_call(
      body,
      grid=(N // RB,),
      in_specs=[
          pl.BlockSpec((RB, D_IN), row),
          pl.BlockSpec((1, RB, D_IN), core0),
          pl.BlockSpec((1, RB, D_IN), core1),
          pl.BlockSpec((1, RB, 16), core0),
          pl.BlockSpec((1, RB, 16), core1),
          pl.BlockSpec((D_IN, D_HID), fix),
          pl.BlockSpec((D_IN, D_HID), fix),
          pl.BlockSpec((1, D_HID), fix),
          pl.BlockSpec((D_HID, CP), fix),
          pl.BlockSpec((D_HID, CP), fix),
          pl.BlockSpec((1, CP), fix),
      ],
      out_specs=[pl.BlockSpec((RB, CP), row), pl.BlockSpec((RB, CP), row)],
      out_shape=[jax.ShapeDtypeStruct((N, CP), jnp.float32)] * 2,
  )(x, sums, sums, cnts, cnts, ws1, wn1, b1, ws2, wn2, b2)


def _tc_final(svec, sums2, cnts):
  def body(s_r, a0_r, a1_r, c0_r, c1_r, o_r):
    deg = c0_r[0][:, 0:1] + c1_r[0][:, 0:1]
    dinv = 1.0 / jnp.maximum(deg, 1.0)
    o_r[...] = (s_r[...] + (a0_r[0] + a1_r[0]) * dinv)[:, :N_CLASSES]

  row = lambda i: (i, 0)
  core0 = lambda i: (0, i, 0)
  core1 = lambda i: (1, i, 0)
  return pl.pallas_call(
      body,
      grid=(N // RB,),
      in_specs=[
          pl.BlockSpec((RB, CP), row),
          pl.BlockSpec((1, RB, CP), core0),
          pl.BlockSpec((1, RB, CP), core1),
          pl.BlockSpec((1, RB, 16), core0),
          pl.BlockSpec((1, RB, 16), core1),
      ],
      out_specs=pl.BlockSpec((RB, N_CLASSES), row),
      out_shape=jax.ShapeDtypeStruct((N, N_CLASSES), jnp.float32),
  )(svec, sums2, sums2, cnts, cnts)


@jax.jit
def kernel(x, edge_index, W_self1, W_neigh1, b1, W_self2, W_neigh2, b2):
  edges = edge_index.reshape(2, EROWS, CHUNK)
  ws2 = jnp.pad(W_self2, ((0, 0), (0, CP - N_CLASSES)))
  wn2 = jnp.pad(W_neigh2, ((0, 0), (0, CP - N_CLASSES)))
  b2p = jnp.pad(b2, (0, CP - N_CLASSES)).reshape(1, CP)

  sums, cnts = _sc_agg_feat(x, edges)
  p, svec = _tc_fused(x, sums, cnts,
                      W_self1, W_neigh1, b1.reshape(1, D_HID), ws2, wn2, b2p)
  sums2 = _sc_agg_proj(p, edges)
  return _tc_final(svec, sums2[0], cnts)


# speedup vs baseline: 1.1982x; 1.1982x over previous
import jax
import jax.numpy as jnp
from jax import lax
from jax.experimental import pallas as pl
from jax.experimental.pallas import tpu as pltpu
from jax.experimental.pallas import tpu_sc as plsc

N = 10000
E = 320000
D_IN = 128
D_HID = 256
N_CLASSES = 41
CP = 48

NC = 2
NS = 16
NW = NC * NS

ROWS_PT = N // NS
RB = 1000

CHUNK = 128
EROWS = E // CHUNK
RPT = EROWS // NW
XTRA = EROWS - RPT * NW
NPAIR = RPT // 2


def _sc_agg(width, with_count):
  mesh = plsc.VectorSubcoreMesh(
      core_axis_name="c", subcore_axis_name="s", num_cores=NC, num_subcores=NS)

  out_type = [jax.ShapeDtypeStruct((NC, N, width), jnp.float32)]
  scratch = [
      pltpu.VMEM((4, CHUNK), jnp.int32),
      pltpu.VMEM((4, CHUNK), jnp.int32),
      pltpu.VMEM((CHUNK, width), jnp.float32),
      pltpu.VMEM((CHUNK, width), jnp.float32),
      pltpu.VMEM_SHARED((N, width), jnp.float32),
      pltpu.SemaphoreType.DMA,
      pltpu.SemaphoreType.DMA,
      pltpu.SemaphoreType.DMA,
  ]
  if with_count:
    out_type.append(jax.ShapeDtypeStruct((NC, N, 16), jnp.float32))
    scratch += [
        pltpu.VMEM((CHUNK, 16), jnp.float32),
        pltpu.VMEM((CHUNK, 16), jnp.float32),
        pltpu.VMEM_SHARED((N, 16), jnp.float32),
    ]
  WL = width // 16

  def body(feat_hbm, edges_hbm, *rest):
    if with_count:
      (sum_hbm, cnt_hbm, sidx, didx, rows_a, rows_b, acc_sh, sem_a, sem_b,
       sem_i, ones_v, z16, cnt_sh) = rest
    else:
      (sum_hbm, sidx, didx, rows_a, rows_b, acc_sh, sem_a, sem_b,
       sem_i) = rest

    c = lax.axis_index("c")
    s = lax.axis_index("s")
    tile = c * NS + s
    base = tile * RPT
    has_extra = tile < XTRA

    def erow(j):
      return jnp.where(j == RPT, EROWS - XTRA + tile,
                       jnp.minimum(base + j, EROWS - 1))

    def idx_load(j, start):
      for ring, half in ((sidx, 0), (didx, 1)):
        cp = pltpu.make_async_copy(edges_hbm.at[half, erow(j)],
                                   ring.at[j % 4], sem_i)
        if start:
          cp.start()
        else:
          cp.wait()

    def scat(rows, slot):
      pltpu.sync_copy(rows, acc_sh.at[didx.at[slot]], add=True)
      if with_count:
        pltpu.sync_copy(ones_v, cnt_sh.at[didx.at[slot]], add=True)

    with jax.named_scope("agg_prologue"):
      def zrow(i, _):
        r = i // WL
        q = (i % WL) * 16
        rows_a[r, pl.ds(q, 16)] = jnp.zeros((16,), jnp.float32)
        return _
      lax.fori_loop(0, CHUNK * WL, zrow, None)
      if with_count:
        def f16(r, _):
          ones_v[r] = jnp.ones((16,), jnp.float32)
          z16[r] = jnp.zeros((16,), jnp.float32)
          return _
        lax.fori_loop(0, CHUNK, f16, None)

      row0 = s * ROWS_PT
      full, part = divmod(ROWS_PT, CHUNK)
      for k in range(full):
        pltpu.sync_copy(rows_a, acc_sh.at[pl.ds(row0 + k * CHUNK, CHUNK)])
        if with_count:
          pltpu.sync_copy(z16, cnt_sh.at[pl.ds(row0 + k * CHUNK, CHUNK)])
      if part:
        pltpu.sync_copy(rows_a.at[pl.ds(0, part)],
                        acc_sh.at[pl.ds(row0 + full * CHUNK, part)])
        if with_count:
          pltpu.sync_copy(z16.at[pl.ds(0, part)],
                          cnt_sh.at[pl.ds(row0 + full * CHUNK, part)])

      for j in range(3):
        pltpu.sync_copy(edges_hbm.at[0, erow(j)], sidx.at[j])
        pltpu.sync_copy(edges_hbm.at[1, erow(j)], didx.at[j])
      plsc.subcore_barrier()

    with jax.named_scope("agg_edges"):
      pltpu.async_copy(feat_hbm.at[sidx.at[0]], rows_a, sem_a)

      def pair(i, _):
        j0 = 2 * i
        s0 = j0 % 4
        s1 = (j0 + 1) % 4

        @pl.when(i > 0)
        def _():
          idx_load(j0 + 1, start=False)
          idx_load(j0 + 2, start=False)

        pltpu.async_copy(feat_hbm.at[sidx.at[s1]], rows_b, sem_b)
        pltpu.make_async_copy(feat_hbm.at[sidx.at[s0]], rows_a, sem_a).wait()
        scat(rows_a, s0)

        idx_load(j0 + 3, start=True)
        idx_load(j0 + 4, start=True)

        @pl.when((i < NPAIR - 1) | has_extra)
        def _():
          pltpu.async_copy(feat_hbm.at[sidx.at[(j0 + 2) % 4]], rows_a, sem_a)

        pltpu.make_async_copy(feat_hbm.at[sidx.at[s1]], rows_b, sem_b).wait()
        scat(rows_b, s1)
        return _
      lax.fori_loop(0, NPAIR, pair, None)

      idx_load(2 * NPAIR + 1, start=False)
      idx_load(2 * NPAIR + 2, start=False)

      @pl.when(has_extra)
      def _():
        pltpu.make_async_copy(feat_hbm.at[sidx.at[RPT % 4]], rows_a,
                              sem_a).wait()
        scat(rows_a, RPT % 4)
      plsc.subcore_barrier()

    with jax.named_scope("agg_writeback"):
      pltpu.sync_copy(acc_sh.at[pl.ds(row0, ROWS_PT)],
                      sum_hbm.at[c, pl.ds(row0, ROWS_PT)])
      if with_count:
        pltpu.sync_copy(cnt_sh.at[pl.ds(row0, ROWS_PT)],
                        cnt_hbm.at[c, pl.ds(row0, ROWS_PT)])

  return pl.kernel(
      body, out_type=out_type, mesh=mesh, scratch_types=scratch,
      compiler_params=pltpu.CompilerParams(use_tc_tiling_on_sc=False))


_sc_agg_feat = _sc_agg(D_IN, with_count=True)
_sc_agg_proj = _sc_agg(CP, with_count=False)


def _tc_fused(x, sums, cnts, ws1, wn1, b1, ws2, wn2, b2):
  def body(x_r, s0_r, s1_r, c0_r, c1_r, ws1_r, wn1_r, b1_r, ws2_r, wn2_r,
           b2_r, p_r, s_r):
    deg = c0_r[0][:, 0:1] + c1_r[0][:, 0:1]
    dinv = 1.0 / jnp.maximum(deg, 1.0)
    a = (s0_r[0] + s1_r[0]) * dinv
    h = (jnp.dot(x_r[...], ws1_r[...], preferred_element_type=jnp.float32)
         + jnp.dot(a, wn1_r[...], preferred_element_type=jnp.float32)
         + b1_r[...])
    h = jnp.maximum(h, 0.0)
    p_r[...] = jnp.dot(h, wn2_r[...], preferred_element_type=jnp.float32)
    s_r[...] = (jnp.dot(h, ws2_r[...], preferred_element_type=jnp.float32)
                + b2_r[...])

  row = lambda i: (i, 0)
  fix = lambda i: (0, 0)
  core0 = lambda i: (0, i, 0)
  core1 = lambda i: (1, i, 0)
  return pl.pallas_call(
      body,
      grid=(N // RB,),
      in_specs=[
          pl.BlockSpec((RB, D_IN), row),
          pl.BlockSpec((1, RB, D_IN), core0),
          pl.BlockSpec((1, RB, D_IN), core1),
          pl.BlockSpec((1, RB, 16), core0),
          pl.BlockSpec((1, RB, 16), core1),
          pl.BlockSpec((D_IN, D_HID), fix),
          pl.BlockSpec((D_IN, D_HID), fix),
          pl.BlockSpec((1, D_HID), fix),
          pl.BlockSpec((D_HID, CP), fix),
          pl.BlockSpec((D_HID, CP), fix),
          pl.BlockSpec((1, CP), fix),
      ],
      out_specs=[pl.BlockSpec((RB, CP), row), pl.BlockSpec((RB, CP), row)],
      out_shape=[jax.ShapeDtypeStruct((N, CP), jnp.float32)] * 2,
  )(x, sums, sums, cnts, cnts, ws1, wn1, b1, ws2, wn2, b2)


def _tc_final(svec, sums2, cnts):
  def body(s_r, a0_r, a1_r, c0_r, c1_r, o_r):
    deg = c0_r[0][:, 0:1] + c1_r[0][:, 0:1]
    dinv = 1.0 / jnp.maximum(deg, 1.0)
    o_r[...] = (s_r[...] + (a0_r[0] + a1_r[0]) * dinv)[:, :N_CLASSES]

  row = lambda i: (i, 0)
  core0 = lambda i: (0, i, 0)
  core1 = lambda i: (1, i, 0)
  return pl.pallas_call(
      body,
      grid=(N // RB,),
      in_specs=[
          pl.BlockSpec((RB, CP), row),
          pl.BlockSpec((1, RB, CP), core0),
          pl.BlockSpec((1, RB, CP), core1),
          pl.BlockSpec((1, RB, 16), core0),
          pl.BlockSpec((1, RB, 16), core1),
      ],
      out_specs=pl.BlockSpec((RB, N_CLASSES), row),
      out_shape=jax.ShapeDtypeStruct((N, N_CLASSES), jnp.float32),
  )(svec, sums2, sums2, cnts, cnts)


@jax.jit
def kernel(x, edge_index, W_self1, W_neigh1, b1, W_self2, W_neigh2, b2):
  edges = edge_index.reshape(2, EROWS, CHUNK)
  ws2 = jnp.pad(W_self2, ((0, 0), (0, CP - N_CLASSES)))
  wn2 = jnp.pad(W_neigh2, ((0, 0), (0, CP - N_CLASSES)))
  b2p = jnp.pad(b2, (0, CP - N_CLASSES)).reshape(1, CP)

  sums, cnts = _sc_agg_feat(x, edges)
  p, svec = _tc_fused(x, sums, cnts,
                      W_self1, W_neigh1, b1.reshape(1, D_HID), ws2, wn2, b2p)
  sums2 = _sc_agg_proj(p, edges)
  return _tc_final(svec, sums2[0], cnts)


# speedup vs baseline: 1.2152x; 1.0141x over previous
import jax
import jax.numpy as jnp
from jax import lax
from jax.experimental import pallas as pl
from jax.experimental.pallas import tpu as pltpu
from jax.experimental.pallas import tpu_sc as plsc

N = 10000
E = 320000
D_IN = 128
D_HID = 256
N_CLASSES = 41
CP = 48

NC = 2
NS = 16
NW = NC * NS

ROWS_PT = N // NS
RB = 2000

CHUNK = 128
EROWS = E // CHUNK
RPT = EROWS // NW
XTRA = EROWS - RPT * NW
NPAIR = RPT // 2


def _sc_agg(width, with_count):
  mesh = plsc.VectorSubcoreMesh(
      core_axis_name="c", subcore_axis_name="s", num_cores=NC, num_subcores=NS)

  out_type = [jax.ShapeDtypeStruct((NC, N, width), jnp.float32)]
  scratch = [
      pltpu.VMEM((4, CHUNK), jnp.int32),
      pltpu.VMEM((4, CHUNK), jnp.int32),
      pltpu.VMEM((CHUNK, width), jnp.float32),
      pltpu.VMEM((CHUNK, width), jnp.float32),
      pltpu.VMEM_SHARED((N, width), jnp.float32),
      pltpu.SemaphoreType.DMA,
      pltpu.SemaphoreType.DMA,
      pltpu.SemaphoreType.DMA,
  ]
  if with_count:
    out_type.append(jax.ShapeDtypeStruct((NC, N, 16), jnp.float32))
    scratch += [
        pltpu.VMEM((CHUNK, 16), jnp.float32),
        pltpu.VMEM((CHUNK, 16), jnp.float32),
        pltpu.VMEM_SHARED((N, 16), jnp.float32),
    ]
  WL = width // 16

  def body(feat_hbm, edges_hbm, *rest):
    if with_count:
      (sum_hbm, cnt_hbm, sidx, didx, rows_a, rows_b, acc_sh, sem_a, sem_b,
       sem_i, ones_v, z16, cnt_sh) = rest
    else:
      (sum_hbm, sidx, didx, rows_a, rows_b, acc_sh, sem_a, sem_b,
       sem_i) = rest

    c = lax.axis_index("c")
    s = lax.axis_index("s")
    tile = c * NS + s
    base = tile * RPT
    has_extra = tile < XTRA

    def erow(j):
      return jnp.where(j == RPT, EROWS - XTRA + tile,
                       jnp.minimum(base + j, EROWS - 1))

    def idx_load(j, start):
      for ring, half in ((sidx, 0), (didx, 1)):
        cp = pltpu.make_async_copy(edges_hbm.at[half, erow(j)],
                                   ring.at[j % 4], sem_i)
        if start:
          cp.start()
        else:
          cp.wait()

    def scat(rows, slot):
      pltpu.sync_copy(rows, acc_sh.at[didx.at[slot]], add=True)
      if with_count:
        pltpu.sync_copy(ones_v, cnt_sh.at[didx.at[slot]], add=True)

    with jax.named_scope("agg_prologue"):
      def zrow(i, _):
        r = i // WL
        q = (i % WL) * 16
        rows_a[r, pl.ds(q, 16)] = jnp.zeros((16,), jnp.float32)
        return _
      lax.fori_loop(0, CHUNK * WL, zrow, None)
      if with_count:
        def f16(r, _):
          ones_v[r] = jnp.ones((16,), jnp.float32)
          z16[r] = jnp.zeros((16,), jnp.float32)
          return _
        lax.fori_loop(0, CHUNK, f16, None)

      row0 = s * ROWS_PT
      full, part = divmod(ROWS_PT, CHUNK)
      for k in range(full):
        pltpu.sync_copy(rows_a, acc_sh.at[pl.ds(row0 + k * CHUNK, CHUNK)])
        if with_count:
          pltpu.sync_copy(z16, cnt_sh.at[pl.ds(row0 + k * CHUNK, CHUNK)])
      if part:
        pltpu.sync_copy(rows_a.at[pl.ds(0, part)],
                        acc_sh.at[pl.ds(row0 + full * CHUNK, part)])
        if with_count:
          pltpu.sync_copy(z16.at[pl.ds(0, part)],
                          cnt_sh.at[pl.ds(row0 + full * CHUNK, part)])

      for j in range(3):
        pltpu.sync_copy(edges_hbm.at[0, erow(j)], sidx.at[j])
        pltpu.sync_copy(edges_hbm.at[1, erow(j)], didx.at[j])
      plsc.subcore_barrier()

    with jax.named_scope("agg_edges"):
      pltpu.async_copy(feat_hbm.at[sidx.at[0]], rows_a, sem_a)

      def pair(i, _):
        j0 = 2 * i
        s0 = j0 % 4
        s1 = (j0 + 1) % 4

        @pl.when(i > 0)
        def _():
          idx_load(j0 + 1, start=False)
          idx_load(j0 + 2, start=False)

        pltpu.async_copy(feat_hbm.at[sidx.at[s1]], rows_b, sem_b)
        pltpu.make_async_copy(feat_hbm.at[sidx.at[s0]], rows_a, sem_a).wait()
        scat(rows_a, s0)

        idx_load(j0 + 3, start=True)
        idx_load(j0 + 4, start=True)

        @pl.when((i < NPAIR - 1) | has_extra)
        def _():
          pltpu.async_copy(feat_hbm.at[sidx.at[(j0 + 2) % 4]], rows_a, sem_a)

        pltpu.make_async_copy(feat_hbm.at[sidx.at[s1]], rows_b, sem_b).wait()
        scat(rows_b, s1)
        return _
      lax.fori_loop(0, NPAIR, pair, None)

      idx_load(2 * NPAIR + 1, start=False)
      idx_load(2 * NPAIR + 2, start=False)

      @pl.when(has_extra)
      def _():
        pltpu.make_async_copy(feat_hbm.at[sidx.at[RPT % 4]], rows_a,
                              sem_a).wait()
        scat(rows_a, RPT % 4)
      plsc.subcore_barrier()

    with jax.named_scope("agg_writeback"):
      pltpu.sync_copy(acc_sh.at[pl.ds(row0, ROWS_PT)],
                      sum_hbm.at[c, pl.ds(row0, ROWS_PT)])
      if with_count:
        pltpu.sync_copy(cnt_sh.at[pl.ds(row0, ROWS_PT)],
                        cnt_hbm.at[c, pl.ds(row0, ROWS_PT)])

  return pl.kernel(
      body, out_type=out_type, mesh=mesh, scratch_types=scratch,
      compiler_params=pltpu.CompilerParams(use_tc_tiling_on_sc=False))


_sc_agg_feat = _sc_agg(D_IN, with_count=True)
_sc_agg_proj = _sc_agg(CP, with_count=False)


def _tc_fused(x, sums, cnts, ws1, wn1, b1, ws2, wn2, b2):
  def body(x_r, s0_r, s1_r, c0_r, c1_r, ws1_r, wn1_r, b1_r, ws2_r, wn2_r,
           b2_r, p_r, s_r):
    deg = c0_r[0][:, 0:1] + c1_r[0][:, 0:1]
    dinv = 1.0 / jnp.maximum(deg, 1.0)
    a = (s0_r[0] + s1_r[0]) * dinv
    h = (jnp.dot(x_r[...], ws1_r[...], preferred_element_type=jnp.float32)
         + jnp.dot(a, wn1_r[...], preferred_element_type=jnp.float32)
         + b1_r[...])
    h = jnp.maximum(h, 0.0)
    p_r[...] = jnp.dot(h, wn2_r[...], preferred_element_type=jnp.float32)
    s_r[...] = (jnp.dot(h, ws2_r[...], preferred_element_type=jnp.float32)
                + b2_r[...])

  row = lambda i: (i, 0)
  fix = lambda i: (0, 0)
  core0 = lambda i: (0, i, 0)
  core1 = lambda i: (1, i, 0)
  return pl.pallas_call(
      body,
      grid=(N // RB,),
      in_specs=[
          pl.BlockSpec((RB, D_IN), row),
          pl.BlockSpec((1, RB, D_IN), core0),
          pl.BlockSpec((1, RB, D_IN), core1),
          pl.BlockSpec((1, RB, 16), core0),
          pl.BlockSpec((1, RB, 16), core1),
          pl.BlockSpec((D_IN, D_HID), fix),
          pl.BlockSpec((D_IN, D_HID), fix),
          pl.BlockSpec((1, D_HID), fix),
          pl.BlockSpec((D_HID, CP), fix),
          pl.BlockSpec((D_HID, CP), fix),
          pl.BlockSpec((1, CP), fix),
      ],
      out_specs=[pl.BlockSpec((RB, CP), row), pl.BlockSpec((RB, CP), row)],
      out_shape=[jax.ShapeDtypeStruct((N, CP), jnp.float32)] * 2,
  )(x, sums, sums, cnts, cnts, ws1, wn1, b1, ws2, wn2, b2)


def _tc_final(svec, sums2, cnts):
  def body(s_r, a0_r, a1_r, c0_r, c1_r, o_r):
    deg = c0_r[0][:, 0:1] + c1_r[0][:, 0:1]
    dinv = 1.0 / jnp.maximum(deg, 1.0)
    o_r[...] = (s_r[...] + (a0_r[0] + a1_r[0]) * dinv)[:, :N_CLASSES]

  row = lambda i: (i, 0)
  core0 = lambda i: (0, i, 0)
  core1 = lambda i: (1, i, 0)
  return pl.pallas_call(
      body,
      grid=(N // RB,),
      in_specs=[
          pl.BlockSpec((RB, CP), row),
          pl.BlockSpec((1, RB, CP), core0),
          pl.BlockSpec((1, RB, CP), core1),
          pl.BlockSpec((1, RB, 16), core0),
          pl.BlockSpec((1, RB, 16), core1),
      ],
      out_specs=pl.BlockSpec((RB, N_CLASSES), row),
      out_shape=jax.ShapeDtypeStruct((N, N_CLASSES), jnp.float32),
  )(svec, sums2, sums2, cnts, cnts)


@jax.jit
def kernel(x, edge_index, W_self1, W_neigh1, b1, W_self2, W_neigh2, b2):
  edges = edge_index.reshape(2, EROWS, CHUNK)
  ws2 = jnp.pad(W_self2, ((0, 0), (0, CP - N_CLASSES)))
  wn2 = jnp.pad(W_neigh2, ((0, 0), (0, CP - N_CLASSES)))
  b2p = jnp.pad(b2, (0, CP - N_CLASSES)).reshape(1, CP)

  sums, cnts = _sc_agg_feat(x, edges)
  p, svec = _tc_fused(x, sums, cnts,
                      W_self1, W_neigh1, b1.reshape(1, D_HID), ws2, wn2, b2p)
  sums2 = _sc_agg_proj(p, edges)
  return _tc_final(svec, sums2[0], cnts)
